# baseline (device time: 19594 ns/iter reference)
import jax
import jax.numpy as jnp
from jax import lax
from jax.experimental import pallas as pl
from jax.experimental.pallas import tpu as pltpu

N_DEV = 4
N_BLK = 8


def kernel(x):
    m_per, n = x.shape
    blk = m_per // N_BLK

    def body(x_ref, out_ref, acc_ref, gather_ref, send_sems, recv_sems):
        j = pl.program_id(0)
        my = lax.axis_index("i")

        chunk = x_ref[...]
        row = lax.broadcasted_iota(jnp.int32, (blk, n), 0)
        grow = row + (my * m_per + j * blk)
        cmax = jnp.max(chunk, axis=0, keepdims=True)
        big_i = jnp.int32(2 * N_DEV * m_per)
        cidx = jnp.min(
            jnp.where(chunk == cmax, grow, big_i), axis=0, keepdims=True
        ).astype(jnp.float32)

        @pl.when(j == 0)
        def _():
            acc_ref[0:1, :] = cmax
            acc_ref[1:2, :] = cidx

        @pl.when(j > 0)
        def _():
            run_max = acc_ref[0:1, :]
            run_idx = acc_ref[1:2, :]
            better = cmax > run_max
            acc_ref[0:1, :] = jnp.where(better, cmax, run_max)
            acc_ref[1:2, :] = jnp.where(better, cidx, run_idx)

        @pl.when(j == N_BLK - 1)
        def _():
            barrier_sem = pltpu.get_barrier_semaphore()
            for k in range(1, N_DEV):
                pl.semaphore_signal(
                    barrier_sem,
                    inc=1,
                    device_id=((my + k) % N_DEV,),
                    device_id_type=pl.DeviceIdType.MESH,
                )

            gather_ref[pl.ds(my, 1), :, :] = acc_ref[...].reshape(1, 2, n)

            pl.semaphore_wait(barrier_sem, N_DEV - 1)

            sends = []
            for k in range(1, N_DEV):
                peer = (my + k) % N_DEV
                d = pltpu.make_async_remote_copy(
                    src_ref=gather_ref.at[pl.ds(my, 1)],
                    dst_ref=gather_ref.at[pl.ds(my, 1)],
                    send_sem=send_sems.at[k - 1],
                    recv_sem=recv_sems.at[k - 1],
                    device_id=(peer,),
                    device_id_type=pl.DeviceIdType.MESH,
                )
                d.start()
                sends.append(d)
            for k in range(1, N_DEV):
                src = (my - k) % N_DEV
                r = pltpu.make_async_remote_copy(
                    src_ref=gather_ref.at[pl.ds(src, 1)],
                    dst_ref=gather_ref.at[pl.ds(src, 1)],
                    send_sem=send_sems.at[k - 1],
                    recv_sem=recv_sems.at[k - 1],
                    device_id=(src,),
                    device_id_type=pl.DeviceIdType.MESH,
                )
                r.wait_recv()
            for d in sends:
                d.wait_send()

            vals = gather_ref[:, 0, :]
            idxs = gather_ref[:, 1, :]
            vmax = jnp.max(vals, axis=0, keepdims=True)
            big_f = jnp.float32(2 * N_DEV * m_per)
            imin = jnp.min(
                jnp.where(vals == vmax, idxs, big_f), axis=0, keepdims=True
            )
            out_ref[0:1, :] = vmax
            out_ref[1:2, :] = imin

    return pl.pallas_call(
        body,
        grid=(N_BLK,),
        in_specs=[pl.BlockSpec((blk, n), lambda j: (j, 0))],
        out_specs=pl.BlockSpec((2, n), lambda j: (0, 0)),
        out_shape=jax.ShapeDtypeStruct((2, n), jnp.float32),
        scratch_shapes=[
            pltpu.VMEM((2, n), jnp.float32),
            pltpu.VMEM((N_DEV, 2, n), jnp.float32),
            pltpu.SemaphoreType.DMA((N_DEV - 1,)),
            pltpu.SemaphoreType.DMA((N_DEV - 1,)),
        ],
        compiler_params=pltpu.CompilerParams(collective_id=0),
    )(x)


# device time: 18981 ns/iter; 1.0323x vs baseline; 1.0323x over previous
import jax
import jax.numpy as jnp
from jax import lax
from jax.experimental import pallas as pl
from jax.experimental.pallas import tpu as pltpu

N_DEV = 4
N_BLK = 8
DEPTH = 4


def kernel(x):
    m_per, n = x.shape
    blk = m_per // N_BLK
    n2 = n // 2

    def body(x_hbm, out_ref, buf, dsems, gather_ref, send_sems, recv_sems):
        my = lax.axis_index("i")
        big_i = jnp.int32(2 * N_DEV * m_per)
        big_f = jnp.float32(2 * N_DEV * m_per)

        copies = []
        for j in range(min(DEPTH, N_BLK)):
            c = pltpu.make_async_copy(
                x_hbm.at[pl.ds(j * blk, blk)], buf.at[j % DEPTH], dsems.at[j % DEPTH]
            )
            c.start()
            copies.append(c)

        barrier_sem = pltpu.get_barrier_semaphore()
        for k in range(1, N_DEV):
            pl.semaphore_signal(
                barrier_sem,
                inc=1,
                device_id=((my + k) % N_DEV,),
                device_id_type=pl.DeviceIdType.MESH,
            )

        def exchange(h):
            sends, recvs = [], []
            for k in range(1, N_DEV):
                peer = (my + k) % N_DEV
                d = pltpu.make_async_remote_copy(
                    src_ref=gather_ref.at[pl.ds(my, 1), :, pl.ds(h * n2, n2)],
                    dst_ref=gather_ref.at[pl.ds(my, 1), :, pl.ds(h * n2, n2)],
                    send_sem=send_sems.at[h, k - 1],
                    recv_sem=recv_sems.at[h, k - 1],
                    device_id=(peer,),
                    device_id_type=pl.DeviceIdType.MESH,
                )
                d.start()
                sends.append(d)
                src = (my - k) % N_DEV
                r = pltpu.make_async_remote_copy(
                    src_ref=gather_ref.at[pl.ds(src, 1), :, pl.ds(h * n2, n2)],
                    dst_ref=gather_ref.at[pl.ds(src, 1), :, pl.ds(h * n2, n2)],
                    send_sem=send_sems.at[h, k - 1],
                    recv_sem=recv_sems.at[h, k - 1],
                    device_id=(src,),
                    device_id_type=pl.DeviceIdType.MESH,
                )
                recvs.append(r)
            return sends, recvs

        def combine_store(h):
            cols = pl.ds(h * n2, n2)
            vals = gather_ref[:, 0, cols]
            idxs = gather_ref[:, 1, cols]
            vmax = jnp.max(vals, axis=0, keepdims=True)
            imin = jnp.min(
                jnp.where(vals == vmax, idxs, big_f), axis=0, keepdims=True
            )
            out_ref[0:1, cols] = vmax
            out_ref[1:2, cols] = imin

        run_max = [None, None]
        run_idx = [None, None]
        pending = {}
        for j in range(N_BLK):
            copies[j].wait()
            if j + DEPTH < N_BLK:
                c = pltpu.make_async_copy(
                    x_hbm.at[pl.ds((j + DEPTH) * blk, blk)],
                    buf.at[(j + DEPTH) % DEPTH],
                    dsems.at[(j + DEPTH) % DEPTH],
                )
                c.start()
                copies.append(c)
            base = my * m_per + j * blk
            for h in (0, 1):
                chunk = buf[j % DEPTH, :, h * n2 : (h + 1) * n2]
                grow = lax.broadcasted_iota(jnp.int32, (blk, n2), 0) + base
                cmax = jnp.max(chunk, axis=0, keepdims=True)
                cidx = jnp.min(
                    jnp.where(chunk == cmax, grow, big_i), axis=0, keepdims=True
                ).astype(jnp.float32)
                if j == 0:
                    run_max[h], run_idx[h] = cmax, cidx
                else:
                    better = cmax > run_max[h]
                    run_max[h] = jnp.where(better, cmax, run_max[h])
                    run_idx[h] = jnp.where(better, cidx, run_idx[h])
                if j == N_BLK - 1:
                    cols = pl.ds(h * n2, n2)
                    gather_ref[pl.ds(my, 1), 0:1, cols] = run_max[h].reshape(1, 1, n2)
                    gather_ref[pl.ds(my, 1), 1:2, cols] = run_idx[h].reshape(1, 1, n2)
                    if h == 0:
                        pl.semaphore_wait(barrier_sem, N_DEV - 1)
                    pending[h] = exchange(h)

        all_sends = []
        for h in (0, 1):
            sends, recvs = pending[h]
            for r in recvs:
                r.wait_recv()
            combine_store(h)
            all_sends += sends
        for d in all_sends:
            d.wait_send()

    return pl.pallas_call(
        body,
        in_specs=[pl.BlockSpec(memory_space=pl.ANY)],
        out_specs=pl.BlockSpec(memory_space=pltpu.VMEM),
        out_shape=jax.ShapeDtypeStruct((2, n), jnp.float32),
        scratch_shapes=[
            pltpu.VMEM((DEPTH, blk, n), jnp.float32),
            pltpu.SemaphoreType.DMA((DEPTH,)),
            pltpu.VMEM((N_DEV, 2, n), jnp.float32),
            pltpu.SemaphoreType.DMA((2, N_DEV - 1)),
            pltpu.SemaphoreType.DMA((2, N_DEV - 1)),
        ],
        compiler_params=pltpu.CompilerParams(
            collective_id=0, vmem_limit_bytes=56 * 1024 * 1024
        ),
    )(x)
